# baseline (speedup 1.0000x reference)
import functools

import jax
import jax.numpy as jnp
from jax import lax
from jax.experimental import pallas as pl
from jax.experimental.pallas import tpu as pltpu
from jax.experimental.pallas import tpu_sc as plsc

NC, NS, L = 2, 16, 16
NW = NC * NS
CH = 16


def _make_kernel(N, D, V):
    b_per_w = N // NW
    sizes = [8, 8] + [16] * ((b_per_w - 32) // 16) + [8, 8]
    offs = [0]
    for s in sizes:
        offs.append(offs[-1] + s)
    assert offs[-1] == b_per_w
    n_ch = len(sizes)
    n_mid = n_ch - 4
    assert n_mid % 2 == 0 and sizes[2:2 + n_mid] == [16] * n_mid
    mesh = plsc.VectorSubcoreMesh(core_axis_name="c", subcore_axis_name="s")

    @functools.partial(
        pl.kernel,
        mesh=mesh,
        out_type=jax.ShapeDtypeStruct((N, D), jnp.float32),
        scratch_types=(
            [pltpu.VMEM((b_per_w,), jnp.int32)]
            + [pltpu.VMEM((CH, D), jnp.float32)] * 6
            + [pltpu.SemaphoreType.DMA] * 6
        ),
    )
    def k(x_hbm, idx_hbm, emb_hbm, out_hbm, idx_v,
          r0, r1, x0, x1, o0, o1, gs0, gs1, xs0, xs1, ws0, ws1):
        rows = [r0, r1]
        xv = [x0, x1]
        ov = [o0, o1]
        gsem = [gs0, gs1]
        xsem = [xs0, xs1]
        wsem = [ws0, ws1]

        wid = lax.axis_index("s") * NC + lax.axis_index("c")
        base = wid * b_per_w
        pltpu.sync_copy(idx_hbm.at[pl.ds(base, b_per_w)], idx_v)

        def gather_cp(off, sz, b):
            return pltpu.make_async_copy(
                emb_hbm.at[idx_v.at[pl.ds(off, sz)]],
                rows[b].at[pl.ds(0, sz)], gsem[b])

        def xload_cp(off, sz, b):
            return pltpu.make_async_copy(
                x_hbm.at[pl.ds(base + off, sz)],
                xv[b].at[pl.ds(0, sz)], xsem[b])

        def wb_cp(off, sz, b):
            return pltpu.make_async_copy(
                ov[b].at[pl.ds(0, sz)],
                out_hbm.at[pl.ds(base + off, sz)], wsem[b])

        def issue_loads(off, sz, b):
            gather_cp(off, sz, b).start()
            xload_cp(off, sz, b).start()

        def add_and_wb(off, sz, prev_off, prev_sz, b, first):
            if not first:
                wb_cp(prev_off, prev_sz, b).wait()
            gather_cp(off, sz, b).wait()
            xload_cp(off, sz, b).wait()

            def row_body(r, rc):
                for dcol in range(D // L):
                    sl = pl.ds(dcol * L, L)
                    ov[b][r, sl] = rows[b][r, sl] + xv[b][r, sl]
                return rc

            lax.fori_loop(0, sz, row_body, 0)
            wb_cp(off, sz, b).start()

        issue_loads(offs[0], sizes[0], 0)
        issue_loads(offs[1], sizes[1], 1)
        add_and_wb(offs[0], sizes[0], 0, 0, 0, True)
        issue_loads(offs[2], sizes[2], 0)
        add_and_wb(offs[1], sizes[1], 0, 0, 1, True)
        issue_loads(offs[3], sizes[3], 1)

        mid0 = offs[2]
        add_and_wb(offs[2], 16, offs[0], 8, 0, False)
        issue_loads(offs[4], 16, 0)
        add_and_wb(offs[3], 16, offs[1], 8, 1, False)
        issue_loads(offs[5], 16, 1)

        def outer(i, carry):
            for j in range(2):
                m = 2 + i * 2 + j
                off = mid0 + m * 16
                b = j
                wb_cp(off - 32, 16, b).wait()
                gather_cp(off, 16, b).wait()
                xload_cp(off, 16, b).wait()

                def row_body(r, rc):
                    for dcol in range(D // L):
                        sl = pl.ds(dcol * L, L)
                        ov[b][r, sl] = rows[b][r, sl] + xv[b][r, sl]
                    return rc

                lax.fori_loop(0, 16, row_body, 0)
                wb_cp(off, 16, b).start()

                @pl.when(m + 2 < n_mid)
                def _():
                    gather_cp(off + 32, 16, b).start()
                    xload_cp(off + 32, 16, b).start()
            return carry

        lax.fori_loop(0, (n_mid - 2) // 2, outer, 0)

        issue_loads(offs[n_ch - 2], 8, 0)
        issue_loads(offs[n_ch - 1], 8, 1)
        add_and_wb(offs[n_ch - 2], 8, offs[n_ch - 4], 16, 0, False)
        add_and_wb(offs[n_ch - 1], 8, offs[n_ch - 3], 16, 1, False)
        wb_cp(offs[n_ch - 2], 8, 0).wait()
        wb_cp(offs[n_ch - 1], 8, 1).wait()

    return k


def kernel(x, tokens, emb):
    B, S, D = x.shape
    V = emb.shape[0]
    N = B * S
    xf = x.reshape(N, D)
    tok = tokens.reshape(N).astype(jnp.int32)
    out = _make_kernel(N, D, V)(xf, tok, emb)
    return out.reshape(B, S, D)
